# baseline (speedup 1.0000x reference)
import jax
import jax.numpy as jnp
from jax.experimental import pallas as pl
from jax.experimental.pallas import tpu as pltpu


def _ms_cam_kernel(x_ref, w1_ref, b1_ref, w2_ref, b2_ref,
                   g1_ref, gb1_ref, g2_ref, gb2_ref, o_ref):
    x = x_ref[...]
    C, HW = x.shape
    xb = x.astype(jnp.bfloat16)

    m = jnp.sum(x, axis=1, keepdims=True) * (1.0 / HW)
    mb = jnp.broadcast_to(m, (C, 128)).astype(jnp.bfloat16)
    hg = jnp.maximum(
        jnp.dot(g1_ref[...], mb, preferred_element_type=jnp.float32)
        + gb1_ref[...], 0.0).astype(jnp.bfloat16)
    xg = (jnp.dot(g2_ref[...], hg, preferred_element_type=jnp.float32)
          + gb2_ref[...])[:, 0:1]

    h = jnp.maximum(
        jnp.dot(w1_ref[...], xb, preferred_element_type=jnp.float32)
        + b1_ref[...], 0.0).astype(jnp.bfloat16)
    xl = (jnp.dot(w2_ref[...], h, preferred_element_type=jnp.float32)
          + b2_ref[...])

    gate = 0.5 + 0.5 * jnp.tanh((xl + xg) * 0.5)
    o_ref[...] = (x * gate).astype(o_ref.dtype)


def kernel(x_nchw, w1, b1, w2, b2, g1, gb1, g2, gb2):
    N, C, H, W = x_nchw.shape
    HW = H * W
    Ci = w1.shape[1]

    x = x_nchw.reshape(N, C, HW)

    w1t = w1.T.astype(jnp.bfloat16)
    b1c = b1.reshape(Ci, 1)
    w2t = w2.T.astype(jnp.bfloat16)
    b2c = b2.reshape(C, 1)
    g1t = g1.T.astype(jnp.bfloat16)
    gb1c = gb1.reshape(Ci, 1)
    g2t = g2.T.astype(jnp.bfloat16)
    gb2c = gb2.reshape(C, 1)

    const = lambda shape: pl.BlockSpec(shape, lambda n: (0,) * len(shape))
    out = pl.pallas_call(
        _ms_cam_kernel,
        out_shape=jax.ShapeDtypeStruct((N, C, HW), x.dtype),
        grid=(N,),
        in_specs=[
            pl.BlockSpec((None, C, HW), lambda n: (n, 0, 0)),
            const((Ci, C)), const((Ci, 1)),
            const((C, Ci)), const((C, 1)),
            const((Ci, C)), const((Ci, 1)),
            const((C, Ci)), const((C, 1)),
        ],
        out_specs=pl.BlockSpec((None, C, HW), lambda n: (n, 0, 0)),
        compiler_params=pltpu.CompilerParams(
            dimension_semantics=("parallel",)),
    )(x, w1t, b1c, w2t, b2c, g1t, gb1c, g2t, gb2c)

    return out.reshape(N, C, H, W)


# speedup vs baseline: 1.0007x; 1.0007x over previous
import jax
import jax.numpy as jnp
from jax.experimental import pallas as pl
from jax.experimental.pallas import tpu as pltpu


def _ms_cam_kernel(x_ref, w1_ref, b1_ref, w2_ref, b2_ref,
                   g1_ref, gb1_ref, g2_ref, gb2_ref, o_ref):
    x = x_ref[...]
    C, HW = x.shape

    m = jnp.sum(x, axis=1, keepdims=True) * (1.0 / HW)
    mb = jnp.broadcast_to(m, (C, 128))
    hg = jnp.maximum(
        jnp.dot(g1_ref[...], mb, preferred_element_type=jnp.float32)
        + gb1_ref[...], 0.0)
    xg = (jnp.dot(g2_ref[...], hg, preferred_element_type=jnp.float32)
          + gb2_ref[...])[:, 0:1]

    h = jnp.maximum(
        jnp.dot(w1_ref[...], x, preferred_element_type=jnp.float32)
        + b1_ref[...], 0.0)
    xl = (jnp.dot(w2_ref[...], h, preferred_element_type=jnp.float32)
          + b2_ref[...])

    gate = 0.5 + 0.5 * jnp.tanh((xl + xg) * 0.5)
    o_ref[...] = (x * gate).astype(o_ref.dtype)


def kernel(x_nchw, w1, b1, w2, b2, g1, gb1, g2, gb2):
    N, C, H, W = x_nchw.shape
    HW = H * W
    Ci = w1.shape[1]

    x = x_nchw.reshape(N, C, HW)

    w1t = w1.T
    b1c = b1.reshape(Ci, 1)
    w2t = w2.T
    b2c = b2.reshape(C, 1)
    g1t = g1.T
    gb1c = gb1.reshape(Ci, 1)
    g2t = g2.T
    gb2c = gb2.reshape(C, 1)

    const = lambda shape: pl.BlockSpec(shape, lambda n: (0,) * len(shape))
    out = pl.pallas_call(
        _ms_cam_kernel,
        out_shape=jax.ShapeDtypeStruct((N, C, HW), x.dtype),
        grid=(N,),
        in_specs=[
            pl.BlockSpec((None, C, HW), lambda n: (n, 0, 0)),
            const((Ci, C)), const((Ci, 1)),
            const((C, Ci)), const((C, 1)),
            const((Ci, C)), const((Ci, 1)),
            const((C, Ci)), const((C, 1)),
        ],
        out_specs=pl.BlockSpec((None, C, HW), lambda n: (n, 0, 0)),
        compiler_params=pltpu.CompilerParams(
            dimension_semantics=("parallel",)),
    )(x, w1t, b1c, w2t, b2c, g1t, gb1c, g2t, gb2c)

    return out.reshape(N, C, H, W)


# speedup vs baseline: 1.0486x; 1.0478x over previous
import jax
import jax.numpy as jnp
from jax.experimental import pallas as pl
from jax.experimental.pallas import tpu as pltpu


def _make_ms_cam_kernel(Ci):
    def _ms_cam_kernel(x_ref, w_ref, b_ref, o_ref):
        x = x_ref[...]
        C, HW = x.shape
        r0 = 2 * Ci

        m = jnp.sum(x, axis=1, keepdims=True) * (1.0 / HW)
        mb = jnp.broadcast_to(m, (C, 128))
        hg = jnp.maximum(
            jnp.dot(w_ref[Ci:r0, :], mb, preferred_element_type=jnp.float32)
            + b_ref[Ci:r0, :], 0.0)
        xg = jnp.dot(w_ref[r0:, Ci:r0], hg,
                     preferred_element_type=jnp.float32)[:, 0:1]

        h = jnp.maximum(
            jnp.dot(w_ref[0:Ci, :], x, preferred_element_type=jnp.float32)
            + b_ref[0:Ci, :], 0.0)
        xl = jnp.dot(w_ref[r0:, 0:Ci], h,
                     preferred_element_type=jnp.float32)

        z = xl + (xg + b_ref[r0:, :])
        gate = 0.5 + 0.5 * jnp.tanh(z * 0.5)
        o_ref[...] = (x * gate).astype(o_ref.dtype)
    return _ms_cam_kernel


def kernel(x_nchw, w1, b1, w2, b2, g1, gb1, g2, gb2):
    N, C, H, W = x_nchw.shape
    HW = H * W
    Ci = w1.shape[1]

    x = x_nchw.reshape(N, C, HW)

    bot = jnp.concatenate([w2.T, g2.T], axis=1)
    if 2 * Ci < C:
        bot = jnp.pad(bot, ((0, 0), (0, C - 2 * Ci)))
    wpack = jnp.concatenate([w1.T, g1.T, bot], axis=0)
    bpack = jnp.concatenate([b1, gb1, b2 + gb2]).reshape(2 * Ci + C, 1)

    const = lambda shape: pl.BlockSpec(shape, lambda n: (0,) * len(shape))
    out = pl.pallas_call(
        _make_ms_cam_kernel(Ci),
        out_shape=jax.ShapeDtypeStruct((N, C, HW), x.dtype),
        grid=(N,),
        in_specs=[
            pl.BlockSpec((None, C, HW), lambda n: (n, 0, 0)),
            const((2 * Ci + C, C)),
            const((2 * Ci + C, 1)),
        ],
        out_specs=pl.BlockSpec((None, C, HW), lambda n: (n, 0, 0)),
        compiler_params=pltpu.CompilerParams(
            dimension_semantics=("parallel",)),
    )(x, wpack, bpack)

    return out.reshape(N, C, H, W)


# speedup vs baseline: 1.0698x; 1.0202x over previous
import jax
import jax.numpy as jnp
from jax.experimental import pallas as pl
from jax.experimental.pallas import tpu as pltpu


def _make_ms_cam_kernel(Ci, NB):
    def _ms_cam_kernel(x_ref, w_ref, b_ref, o_ref):
        r0 = 2 * Ci
        for i in range(NB):
            x = x_ref[i]
            C, HW = x.shape

            m = jnp.sum(x, axis=1, keepdims=True) * (1.0 / HW)
            mb = jnp.broadcast_to(m, (C, 128))
            hg = jnp.maximum(
                jnp.dot(w_ref[Ci:r0, :], mb, preferred_element_type=jnp.float32)
                + b_ref[Ci:r0, :], 0.0)
            xg = jnp.dot(w_ref[r0:, Ci:r0], hg,
                         preferred_element_type=jnp.float32)[:, 0:1]

            h = jnp.maximum(
                jnp.dot(w_ref[0:Ci, :], x, preferred_element_type=jnp.float32)
                + b_ref[0:Ci, :], 0.0)
            xl = jnp.dot(w_ref[r0:, 0:Ci], h,
                         preferred_element_type=jnp.float32)

            z = xl + (xg + b_ref[r0:, :])
            gate = 0.5 + 0.5 * jnp.tanh(z * 0.5)
            o_ref[i] = (x * gate).astype(o_ref.dtype)
    return _ms_cam_kernel


def kernel(x_nchw, w1, b1, w2, b2, g1, gb1, g2, gb2):
    N, C, H, W = x_nchw.shape
    HW = H * W
    Ci = w1.shape[1]

    x = x_nchw.reshape(N, C, HW)

    bot = jnp.concatenate([w2.T, g2.T], axis=1)
    if 2 * Ci < C:
        bot = jnp.pad(bot, ((0, 0), (0, C - 2 * Ci)))
    wpack = jnp.concatenate([w1.T, g1.T, bot], axis=0)
    bpack = jnp.concatenate([b1, gb1, b2 + gb2]).reshape(2 * Ci + C, 1)

    NB = 2 if N % 2 == 0 else 1
    const = lambda shape: pl.BlockSpec(shape, lambda n: (0,) * len(shape))
    out = pl.pallas_call(
        _make_ms_cam_kernel(Ci, NB),
        out_shape=jax.ShapeDtypeStruct((N, C, HW), x.dtype),
        grid=(N // NB,),
        in_specs=[
            pl.BlockSpec((NB, C, HW), lambda n: (n, 0, 0)),
            const((2 * Ci + C, C)),
            const((2 * Ci + C, 1)),
        ],
        out_specs=pl.BlockSpec((NB, C, HW), lambda n: (n, 0, 0)),
        compiler_params=pltpu.CompilerParams(
            dimension_semantics=("parallel",)),
    )(x, wpack, bpack)

    return out.reshape(N, C, H, W)
